# baseline (speedup 1.0000x reference)
import functools

import jax
import jax.numpy as jnp
from jax import lax
from jax.experimental import pallas as pl
from jax.experimental.pallas import tpu as pltpu
from jax.experimental.pallas import tpu_sc as plsc

_VOCAB = 100000
_EMB = 32
_NCAT = 20
_NCONT = 6
_NFEAT = _NCAT + _NCONT

_NW = 32
_CH = 64
_KG = _CH * _NCAT // 128
_KC = _CH * _NCONT // 128
_PB = 32


def _prep_body(x_ref, out_ref):
    t = x_ref.shape[1]
    offs = lax.broadcasted_iota(jnp.int32, (t, _NCAT), 1) * _VOCAB
    for bb in range(_PB):
        cat = x_ref[bb, :, 0:_NCAT].astype(jnp.int32) + offs
        out_ref[pl.ds(bb * t, t), 0:_NCAT] = lax.bitcast_convert_type(
            cat, jnp.float32)
        out_ref[pl.ds(bb * t, t), _NCAT:_NFEAT] = x_ref[bb, :, _NCAT:_NFEAT]


def _build_sc(bt):
    pos_per_w = bt // _NW
    n_chunks = pos_per_w // _CH
    mesh = plsc.VectorSubcoreMesh(core_axis_name="c", subcore_axis_name="s")

    @functools.partial(
        pl.kernel,
        mesh=mesh,
        compiler_params=pltpu.CompilerParams(
            use_tc_tiling_on_sc=False, needs_layout_passes=False),
        out_type=jax.ShapeDtypeStruct((bt * _NFEAT, _EMB), jnp.float32),
        scratch_types=[
            pltpu.VMEM((_CH, 32), jnp.float32),
            pltpu.VMEM((_CH * _NCAT,), jnp.int32),
            pltpu.VMEM((_CH * _NCAT,), jnp.int32),
            pltpu.VMEM((_CH * _NCONT,), jnp.int32),
            pltpu.VMEM((_CH * _NCAT, _EMB), jnp.float32),
            pltpu.VMEM((_CH * _NCONT, _EMB), jnp.float32),
            pltpu.VMEM((_NCONT * _EMB,), jnp.float32),
            pltpu.VMEM((_NCONT * _EMB,), jnp.float32),
            pltpu.SemaphoreType.DMA,
            pltpu.SemaphoreType.DMA,
        ],
    )
    def k(xmix_hbm, table_hbm, wc_hbm, bc_hbm, out_hbm,
          xv, idxv, dstv, cdstv, rows, contv, wcv, bcv, gsem, ssem):
        wid = lax.axis_index("s") * 2 + lax.axis_index("c")
        pltpu.sync_copy(wc_hbm, wcv)
        pltpu.sync_copy(bc_hbm, bcv)
        base_pos = wid * pos_per_w
        iota16 = lax.broadcasted_iota(jnp.int32, (16,), 0)

        def chunk_body(c, carry):
            p0 = base_pos + c * _CH
            o0 = p0 * _NFEAT
            pltpu.sync_copy(xmix_hbm.at[pl.ds(p0, _CH), :], xv)

            def blk_body(blk, _):
                i = iota16 + blk * 16
                d = (i.astype(jnp.float32) * (1.0 / _NCAT)).astype(jnp.int32)
                m = i - d * _NCAT
                gi = plsc.load_gather(xv, [d, m])
                idxv[pl.ds(blk * 16, 16)] = plsc.bitcast(gi, jnp.int32)
                dstv[pl.ds(blk * 16, 16)] = (
                    o0 + i + (_NFEAT - _NCAT) * d)
                return 0

            lax.fori_loop(0, _CH * _NCAT // 16, blk_body, 0)

            def cblk_body(blk, _):
                r = iota16 + blk * 16
                d = (r.astype(jnp.float32) * (1.0 / _NCONT)).astype(jnp.int32)
                cdstv[pl.ds(blk * 16, 16)] = (
                    o0 + _NCAT + r + _NCAT * d)
                return 0

            lax.fori_loop(0, _CH * _NCONT // 16, cblk_body, 0)

            gh = pltpu.async_copy(table_hbm.at[idxv], rows, gsem)

            for f in range(_NCONT):
                wrows = [wcv[pl.ds(f * _EMB + h * 16, 16)] for h in range(2)]
                brows = [bcv[pl.ds(f * _EMB + h * 16, 16)] for h in range(2)]
                colf = jnp.full((16,), _NCAT + f, jnp.int32)

                def pb_body(pb, _, f=f, wrows=wrows, brows=brows, colf=colf):
                    vec = plsc.load_gather(xv, [iota16 + pb * 16, colf])
                    ridx = iota16 * _NCONT + (pb * 16 * _NCONT + f)
                    for e in range(_EMB):
                        w = wrows[e // 16][e % 16]
                        b = brows[e // 16][e % 16]
                        pre = vec * w + b
                        y = pre / (1.0 + jnp.exp(-pre))
                        cidx = jnp.full((16,), e, jnp.int32)
                        plsc.store_scatter(contv, [ridx, cidx], y)
                    return 0

                lax.fori_loop(0, _CH // 16, pb_body, 0)

            gh.wait()

            sh = [
                pltpu.async_copy(rows, out_hbm.at[dstv], ssem),
                pltpu.async_copy(contv, out_hbm.at[cdstv], ssem),
            ]
            for h in sh:
                h.wait()
            return carry

        lax.fori_loop(0, n_chunks, chunk_body, 0)

    return k


def kernel(x, tables, Wc, bc):
    B, T, _ = x.shape
    ncat, vocab, emb = tables.shape
    bt = B * T
    table2d = tables.reshape(ncat * vocab, emb)

    xmix = pl.pallas_call(
        _prep_body,
        grid=(B // _PB,),
        in_specs=[pl.BlockSpec((_PB, T, _NFEAT), lambda b: (b, 0, 0))],
        out_specs=pl.BlockSpec((_PB * T, 32), lambda b: (b, 0)),
        out_shape=jax.ShapeDtypeStruct((bt, 32), jnp.float32),
    )(x)

    staging = _build_sc(bt)(xmix, table2d, Wc.reshape(-1), bc.reshape(-1))
    return staging.reshape(B, T, _NFEAT, emb)


# speedup vs baseline: 1.1484x; 1.1484x over previous
import functools

import jax
import jax.numpy as jnp
from jax import lax
from jax.experimental import pallas as pl
from jax.experimental.pallas import tpu as pltpu
from jax.experimental.pallas import tpu_sc as plsc

_VOCAB = 100000
_EMB = 32
_NCAT = 20
_NCONT = 6
_NFEAT = _NCAT + _NCONT

_NW = 32
_CH = 64
_KG = _CH * _NCAT // 128
_KC = _CH * _NCONT // 128
_PB = 32


def _prep_body(x_ref, out_ref):
    t = x_ref.shape[1]
    offs = lax.broadcasted_iota(jnp.int32, (t, _NCAT), 1) * _VOCAB
    for bb in range(_PB):
        cat = x_ref[bb, :, 0:_NCAT].astype(jnp.int32) + offs
        out_ref[pl.ds(bb * t, t), 0:_NCAT] = lax.bitcast_convert_type(
            cat, jnp.float32)
        out_ref[pl.ds(bb * t, t), _NCAT:_NFEAT] = x_ref[bb, :, _NCAT:_NFEAT]


def _build_sc(bt):
    pos_per_w = bt // _NW
    n_chunks = pos_per_w // _CH
    n_pairs = n_chunks // 2
    mesh = plsc.VectorSubcoreMesh(core_axis_name="c", subcore_axis_name="s")

    buf = lambda: [
        pltpu.VMEM((_CH, 32), jnp.float32),
        pltpu.VMEM((_CH * _NCAT,), jnp.int32),
        pltpu.VMEM((_CH * _NCAT,), jnp.int32),
        pltpu.VMEM((_CH * _NCONT,), jnp.int32),
        pltpu.VMEM((_CH * _NCAT, _EMB), jnp.float32),
        pltpu.VMEM((_CH * _NCONT, _EMB), jnp.float32),
        pltpu.SemaphoreType.DMA,
    ]

    @functools.partial(
        pl.kernel,
        mesh=mesh,
        compiler_params=pltpu.CompilerParams(
            use_tc_tiling_on_sc=False, needs_layout_passes=False),
        out_type=jax.ShapeDtypeStruct((bt * _NFEAT, _EMB), jnp.float32),
        scratch_types=buf() + buf() + [
            pltpu.VMEM((_NCONT * _EMB,), jnp.float32),
            pltpu.VMEM((_NCONT * _EMB,), jnp.float32),
            pltpu.SemaphoreType.DMA,
        ],
    )
    def k(xmix_hbm, table_hbm, wc_hbm, bc_hbm, out_hbm,
          xv0, idxv0, dstv0, cdstv0, rows0, contv0, ssem0,
          xv1, idxv1, dstv1, cdstv1, rows1, contv1, ssem1,
          wcv, bcv, gsem):
        wid = lax.axis_index("s") * 2 + lax.axis_index("c")
        pltpu.sync_copy(wc_hbm, wcv)
        pltpu.sync_copy(bc_hbm, bcv)
        base_pos = wid * pos_per_w
        iota16 = lax.broadcasted_iota(jnp.int32, (16,), 0)
        bufs = [(xv0, idxv0, dstv0, cdstv0, rows0, contv0, ssem0),
                (xv1, idxv1, dstv1, cdstv1, rows1, contv1, ssem1)]

        def do_chunk(c, b, warm):
            xv, idxv, dstv, cdstv, rows, contv, ssem = bufs[b]
            p0 = base_pos + c * _CH
            o0 = p0 * _NFEAT
            pltpu.sync_copy(xmix_hbm.at[pl.ds(p0, _CH), :], xv)
            if warm:
                pltpu.make_async_copy(rows, out_hbm.at[dstv], ssem).wait()
                pltpu.make_async_copy(contv, out_hbm.at[cdstv], ssem).wait()

            def blk_body(blk, _):
                i = iota16 + blk * 16
                d = (i.astype(jnp.float32) * (1.0 / _NCAT)).astype(jnp.int32)
                m = i - d * _NCAT
                gi = plsc.load_gather(xv, [d, m])
                idxv[pl.ds(blk * 16, 16)] = plsc.bitcast(gi, jnp.int32)
                dstv[pl.ds(blk * 16, 16)] = o0 + i + (_NFEAT - _NCAT) * d
                return 0

            lax.fori_loop(0, _CH * _NCAT // 16, blk_body, 0)

            def cblk_body(blk, _):
                r = iota16 + blk * 16
                d = (r.astype(jnp.float32) * (1.0 / _NCONT)).astype(jnp.int32)
                cdstv[pl.ds(blk * 16, 16)] = o0 + _NCAT + r + _NCAT * d
                return 0

            lax.fori_loop(0, _CH * _NCONT // 16, cblk_body, 0)

            gh = pltpu.async_copy(table_hbm.at[idxv], rows, gsem)

            wv = [[wcv[pl.ds(f * _EMB + h * 16, 16)] for h in range(2)]
                  for f in range(_NCONT)]
            bv = [[bcv[pl.ds(f * _EMB + h * 16, 16)] for h in range(2)]
                  for f in range(_NCONT)]

            def q_body(q, _):
                v = xv[q, pl.ds(16, 16)]
                for f in range(_NCONT):
                    xs = v[4 + f]
                    for h in range(2):
                        pre = wv[f][h] * xs + bv[f][h]
                        contv[q * _NCONT + f, pl.ds(h * 16, 16)] = (
                            pre / (1.0 + jnp.exp(-pre)))
                return 0

            lax.fori_loop(0, _CH, q_body, 0)

            gh.wait()
            pltpu.async_copy(rows, out_hbm.at[dstv], ssem)
            pltpu.async_copy(contv, out_hbm.at[cdstv], ssem)

        do_chunk(0, 0, False)
        do_chunk(1, 1, False)

        def pair_body(k_, carry):
            do_chunk(2 * k_, 0, True)
            do_chunk(2 * k_ + 1, 1, True)
            return carry

        lax.fori_loop(1, n_pairs, pair_body, 0)
        do_chunk(n_chunks - 1, 0, True)

        for b in range(2):
            xv, idxv, dstv, cdstv, rows, contv, ssem = bufs[b]
            pltpu.make_async_copy(rows, out_hbm.at[dstv], ssem).wait()
            pltpu.make_async_copy(contv, out_hbm.at[cdstv], ssem).wait()

    return k


def kernel(x, tables, Wc, bc):
    B, T, _ = x.shape
    ncat, vocab, emb = tables.shape
    bt = B * T
    table2d = tables.reshape(ncat * vocab, emb)

    xmix = pl.pallas_call(
        _prep_body,
        grid=(B // _PB,),
        in_specs=[pl.BlockSpec((_PB, T, _NFEAT), lambda b: (b, 0, 0))],
        out_specs=pl.BlockSpec((_PB * T, 32), lambda b: (b, 0)),
        out_shape=jax.ShapeDtypeStruct((bt, 32), jnp.float32),
    )(x)

    staging = _build_sc(bt)(xmix, table2d, Wc.reshape(-1), bc.reshape(-1))
    return staging.reshape(B, T, _NFEAT, emb)


# speedup vs baseline: 1.1513x; 1.0025x over previous
import functools

import jax
import jax.numpy as jnp
from jax import lax
from jax.experimental import pallas as pl
from jax.experimental.pallas import tpu as pltpu
from jax.experimental.pallas import tpu_sc as plsc

_VOCAB = 100000
_EMB = 32
_NCAT = 20
_NCONT = 6
_NFEAT = _NCAT + _NCONT

_NW = 32
_CH = 64
_KG = _CH * _NCAT // 128
_KC = _CH * _NCONT // 128
_PB = 32


def _prep_body(x_ref, out_ref):
    t = x_ref.shape[1]
    offs = lax.broadcasted_iota(jnp.int32, (t, _NCAT), 1) * _VOCAB
    for bb in range(_PB):
        cat = x_ref[bb, :, 0:_NCAT].astype(jnp.int32) + offs
        out_ref[pl.ds(bb * t, t), 0:_NCAT] = lax.bitcast_convert_type(
            cat, jnp.float32)
        out_ref[pl.ds(bb * t, t), _NCAT:_NFEAT] = x_ref[bb, :, _NCAT:_NFEAT]


def _build_sc(bt):
    pos_per_w = bt // _NW
    n_chunks = pos_per_w // _CH
    n_pairs = n_chunks // 2
    mesh = plsc.VectorSubcoreMesh(core_axis_name="c", subcore_axis_name="s")

    buf = lambda: [
        pltpu.VMEM((_CH, 32), jnp.float32),
        pltpu.VMEM((_CH * _NCAT,), jnp.int32),
        pltpu.VMEM((_CH * _NCAT,), jnp.int32),
        pltpu.VMEM((_CH * _NCONT,), jnp.int32),
        pltpu.VMEM((_CH * _NCAT, _EMB), jnp.float32),
        pltpu.VMEM((_CH * _NCONT, _EMB), jnp.float32),
        pltpu.SemaphoreType.DMA,
        pltpu.SemaphoreType.DMA,
    ]

    @functools.partial(
        pl.kernel,
        mesh=mesh,
        compiler_params=pltpu.CompilerParams(
            use_tc_tiling_on_sc=False, needs_layout_passes=False),
        out_type=jax.ShapeDtypeStruct((bt * _NFEAT, _EMB), jnp.float32),
        scratch_types=buf() + buf() + [
            pltpu.VMEM((_NCONT * _EMB,), jnp.float32),
            pltpu.VMEM((_NCONT * _EMB,), jnp.float32),
            pltpu.SemaphoreType.DMA,
        ],
    )
    def k(xmix_hbm, table_hbm, wc_hbm, bc_hbm, out_hbm,
          xv0, idxv0, dstv0, cdstv0, rows0, contv0, ssem0, xsem0,
          xv1, idxv1, dstv1, cdstv1, rows1, contv1, ssem1, xsem1,
          wcv, bcv, gsem):
        wid = lax.axis_index("s") * 2 + lax.axis_index("c")
        pltpu.sync_copy(wc_hbm, wcv)
        pltpu.sync_copy(bc_hbm, bcv)
        base_pos = wid * pos_per_w
        iota16 = lax.broadcasted_iota(jnp.int32, (16,), 0)
        bufs = [(xv0, idxv0, dstv0, cdstv0, rows0, contv0, ssem0, xsem0),
                (xv1, idxv1, dstv1, cdstv1, rows1, contv1, ssem1, xsem1)]

        def xfetch(c, b):
            xv, xsem = bufs[b][0], bufs[b][7]
            p0 = base_pos + c * _CH
            pltpu.async_copy(xmix_hbm.at[pl.ds(p0, _CH), :], xv, xsem)

        xfetch(0, 0)
        xfetch(1, 1)

        def do_chunk(c, b, warm):
            xv, idxv, dstv, cdstv, rows, contv, ssem, xsem = bufs[b]
            p0 = base_pos + c * _CH
            o0 = p0 * _NFEAT
            pltpu.make_async_copy(
                xmix_hbm.at[pl.ds(p0, _CH), :], xv, xsem).wait()
            if warm:
                pltpu.make_async_copy(rows, out_hbm.at[dstv], ssem).wait()
                pltpu.make_async_copy(contv, out_hbm.at[cdstv], ssem).wait()

            def blk_body(blk, _):
                i = iota16 + blk * 16
                d = (i.astype(jnp.float32) * (1.0 / _NCAT)).astype(jnp.int32)
                m = i - d * _NCAT
                gi = plsc.load_gather(xv, [d, m])
                idxv[pl.ds(blk * 16, 16)] = plsc.bitcast(gi, jnp.int32)
                dstv[pl.ds(blk * 16, 16)] = o0 + i + (_NFEAT - _NCAT) * d
                return 0

            lax.fori_loop(0, _CH * _NCAT // 16, blk_body, 0)

            def cblk_body(blk, _):
                r = iota16 + blk * 16
                d = (r.astype(jnp.float32) * (1.0 / _NCONT)).astype(jnp.int32)
                cdstv[pl.ds(blk * 16, 16)] = o0 + _NCAT + r + _NCAT * d
                return 0

            lax.fori_loop(0, _CH * _NCONT // 16, cblk_body, 0)

            gh = pltpu.async_copy(table_hbm.at[idxv], rows, gsem)

            wv = [[wcv[pl.ds(f * _EMB + h * 16, 16)] for h in range(2)]
                  for f in range(_NCONT)]
            bv = [[bcv[pl.ds(f * _EMB + h * 16, 16)] for h in range(2)]
                  for f in range(_NCONT)]

            def q_body(q, _):
                v = xv[q, pl.ds(16, 16)]
                for f in range(_NCONT):
                    xs = v[4 + f]
                    for h in range(2):
                        pre = wv[f][h] * xs + bv[f][h]
                        contv[q * _NCONT + f, pl.ds(h * 16, 16)] = (
                            pre / (1.0 + jnp.exp(-pre)))
                return 0

            lax.fori_loop(0, _CH, q_body, 0)

            gh.wait()
            pltpu.async_copy(rows, out_hbm.at[dstv], ssem)
            pltpu.async_copy(contv, out_hbm.at[cdstv], ssem)
            cn = jnp.minimum(c + 2, n_chunks - 1)
            pn = base_pos + cn * _CH
            pltpu.async_copy(xmix_hbm.at[pl.ds(pn, _CH), :], xv, xsem)

        do_chunk(0, 0, False)
        do_chunk(1, 1, False)

        def pair_body(k_, carry):
            do_chunk(2 * k_, 0, True)
            do_chunk(2 * k_ + 1, 1, True)
            return carry

        lax.fori_loop(1, n_pairs, pair_body, 0)
        do_chunk(n_chunks - 1, 0, True)

        for b in range(2):
            xv, idxv, dstv, cdstv, rows, contv, ssem, xsem = bufs[b]
            pltpu.make_async_copy(rows, out_hbm.at[dstv], ssem).wait()
            pltpu.make_async_copy(contv, out_hbm.at[cdstv], ssem).wait()
            pltpu.make_async_copy(
                xmix_hbm.at[pl.ds(base_pos, _CH), :], xv, xsem).wait()

    return k


def kernel(x, tables, Wc, bc):
    B, T, _ = x.shape
    ncat, vocab, emb = tables.shape
    bt = B * T
    table2d = tables.reshape(ncat * vocab, emb)

    xmix = pl.pallas_call(
        _prep_body,
        grid=(B // _PB,),
        in_specs=[pl.BlockSpec((_PB, T, _NFEAT), lambda b: (b, 0, 0))],
        out_specs=pl.BlockSpec((_PB * T, 32), lambda b: (b, 0)),
        out_shape=jax.ShapeDtypeStruct((bt, 32), jnp.float32),
    )(x)

    staging = _build_sc(bt)(xmix, table2d, Wc.reshape(-1), bc.reshape(-1))
    return staging.reshape(B, T, _NFEAT, emb)
